# baseline (speedup 1.0000x reference)
import jax
import jax.numpy as jnp
from jax.experimental import pallas as pl
from jax.experimental.pallas import tpu as pltpu


def _copy_kernel(src_ref, x_ref, o_ref):
    del src_ref
    o_ref[0] = x_ref[...]


def kernel(x, t_length):
    N, C, T, H, W = x.shape
    t = jnp.arange(T, dtype=jnp.int32)
    idx = jnp.mod(t_length.astype(jnp.int32) - 1, T)
    mask = jnp.zeros((T,), bool).at[idx].set(True)
    src = jnp.where(mask, t, jnp.minimum(t + 1, T - 1)).astype(jnp.int32)
    srcmap = jnp.concatenate([t, src])

    def in_map(i, n, tt, sref):
        return (n, 0, sref[i * T + tt], 0, 0)

    def out_map(i, n, tt, sref):
        return (i, n, 0, tt, 0, 0)

    out = pl.pallas_call(
        _copy_kernel,
        grid_spec=pltpu.PrefetchScalarGridSpec(
            num_scalar_prefetch=1,
            grid=(2, N, T),
            in_specs=[pl.BlockSpec((1, C, 1, H, W), in_map)],
            out_specs=pl.BlockSpec((1, 1, C, 1, H, W), out_map),
        ),
        out_shape=jax.ShapeDtypeStruct((2, N, C, T, H, W), x.dtype),
    )(srcmap, x)
    return out


# speedup vs baseline: 6.7126x; 6.7126x over previous
import jax
import jax.numpy as jnp
from jax.experimental import pallas as pl
from jax.experimental.pallas import tpu as pltpu


def _shift_kernel(idx_ref, x_ref, o_ref):
    T = x_ref.shape[2]
    o_ref[0] = x_ref[...]
    o_ref[1, :, :, : T - 1] = x_ref[:, :, 1:]
    o_ref[1, :, :, T - 1 :] = x_ref[:, :, T - 1 :]
    for n in range(idx_ref.shape[0]):
        i = idx_ref[n]
        o_ref[1, :, :, pl.ds(i, 1)] = x_ref[:, :, pl.ds(i, 1)]


def kernel(x, t_length):
    N, C, T, H, W = x.shape
    HW = H * W
    Cb = 128
    idx = jnp.mod(t_length.astype(jnp.int32) - 1, T)
    xr = x.reshape(N, C, T, HW)

    def in_map(n, c, iref):
        return (n, c, 0, 0)

    def out_map(n, c, iref):
        return (0, n, c, 0, 0)

    out = pl.pallas_call(
        _shift_kernel,
        grid_spec=pltpu.PrefetchScalarGridSpec(
            num_scalar_prefetch=1,
            grid=(N, C // Cb),
            in_specs=[pl.BlockSpec((1, Cb, T, HW), in_map)],
            out_specs=pl.BlockSpec((2, 1, Cb, T, HW), out_map),
        ),
        out_shape=jax.ShapeDtypeStruct((2, N, C, T, HW), x.dtype),
    )(idx, xr)
    return out.reshape(2, N, C, T, H, W)
